# baseline (device time: 30456 ns/iter reference)
import jax
import jax.numpy as jnp
from jax import lax
from jax.experimental import pallas as pl
from jax.experimental.pallas import tpu as pltpu

N_DEV = 4


def kernel(x, router_W, route_idx, expert_W, shared_W):
    n_tok, d = x.shape
    e_per, _, h = expert_W.shape
    half = e_per // 2

    def body(x_hbm, rw_hbm, idx_hbm, ew_hbm, sw_hbm, out_ref,
             x_ref, rw_ref, idx_ref, ew_ref, sw_ref,
             cw_ref, ccw_ref, cw_send, cw_recv, ccw_send, ccw_recv,
             in_sems):
        my = lax.axis_index("i")
        left = lax.rem(my - 1 + N_DEV, N_DEV)
        right = lax.rem(my + 1, N_DEV)

        cp_ew = pltpu.make_async_copy(ew_hbm, ew_ref, in_sems.at[0])
        cp_x = pltpu.make_async_copy(x_hbm, x_ref, in_sems.at[1])
        cp_rw = pltpu.make_async_copy(rw_hbm, rw_ref, in_sems.at[2])
        cp_idx = pltpu.make_async_copy(idx_hbm, idx_ref, in_sems.at[3])
        cp_sw = pltpu.make_async_copy(sw_hbm, sw_ref, in_sems.at[4])
        cp_ew.start()
        cp_x.start()
        cp_rw.start()
        cp_idx.start()
        cp_sw.start()

        barrier = pltpu.get_barrier_semaphore()
        for nbr in (left, right):
            pl.semaphore_signal(barrier, inc=1, device_id=(nbr,),
                                device_id_type=pl.DeviceIdType.MESH)
        pl.semaphore_wait(barrier, 2)

        rdma_cw = [[None] * half for _ in range(N_DEV - 1)]
        rdma_ccw = [[None] * half for _ in range(N_DEV - 1)]

        def start_cw(hop, sub):
            rdma_cw[hop][sub] = pltpu.make_async_remote_copy(
                src_ref=cw_ref.at[hop, sub],
                dst_ref=cw_ref.at[hop + 1, sub],
                send_sem=cw_send.at[hop, sub],
                recv_sem=cw_recv.at[hop, sub],
                device_id=(right,),
                device_id_type=pl.DeviceIdType.MESH,
            )
            rdma_cw[hop][sub].start()

        def start_ccw(hop, sub):
            rdma_ccw[hop][sub] = pltpu.make_async_remote_copy(
                src_ref=ccw_ref.at[hop, sub],
                dst_ref=ccw_ref.at[hop + 1, sub],
                send_sem=ccw_send.at[hop, sub],
                recv_sem=ccw_recv.at[hop, sub],
                device_id=(left,),
                device_id_type=pl.DeviceIdType.MESH,
            )
            rdma_ccw[hop][sub].start()

        cp_ew.wait()
        cw_ref[0] = ew_ref[:half].astype(jnp.bfloat16)
        ccw_ref[0] = ew_ref[half:].astype(jnp.bfloat16)

        for sub in range(half):
            start_cw(0, sub)
            start_ccw(0, sub)

        cp_x.wait()
        cp_rw.wait()
        cp_idx.wait()
        xv = x_ref[...]
        idx = idx_ref[...]
        scores = jnp.dot(xv, rw_ref[...], preferred_element_type=jnp.float32)
        s_max = jnp.max(scores, axis=-1, keepdims=True)
        e_s = jnp.exp(scores - s_max)
        probs = e_s / jnp.sum(e_s, axis=-1, keepdims=True)
        iota = lax.broadcasted_iota(jnp.int32, scores.shape, 1)
        p_sel = jnp.sum(jnp.where(iota == idx, probs, 0.0), axis=-1,
                        keepdims=True)

        def expert_mm(acc, e_id, w):
            sel = jnp.where(idx == e_id, p_sel, 0.0)
            y = jnp.dot((xv * sel).astype(jnp.bfloat16), w,
                        preferred_element_type=jnp.float32)
            return y if acc is None else acc + y

        acc = None
        for hop in range(N_DEV):
            o_cw = lax.rem(my - hop + N_DEV, N_DEV)
            o_ccw = lax.rem(my + hop, N_DEV)
            for sub in range(half):
                if hop > 0:
                    rdma_cw[hop - 1][sub].wait_recv()
                    rdma_ccw[hop - 1][sub].wait_recv()
                    if hop < N_DEV - 1:
                        start_cw(hop, sub)
                        start_ccw(hop, sub)
                acc = expert_mm(acc, o_cw * e_per + sub, cw_ref[hop, sub])
                acc = expert_mm(acc, o_ccw * e_per + half + sub,
                                ccw_ref[hop, sub])
            if hop == 1:
                cp_sw.wait()
                acc = acc + jnp.dot(xv, sw_ref[...],
                                    preferred_element_type=jnp.float32)
            if hop > 0:
                for sub in range(half):
                    rdma_cw[hop - 1][sub].wait_send()
                    rdma_ccw[hop - 1][sub].wait_send()

        out_ref[...] = acc

    return pl.pallas_call(
        body,
        out_shape=jax.ShapeDtypeStruct((n_tok, h), jnp.float32),
        in_specs=[pl.BlockSpec(memory_space=pl.ANY)] * 5,
        out_specs=pl.BlockSpec(memory_space=pltpu.VMEM),
        scratch_shapes=[
            pltpu.VMEM((n_tok, d), jnp.float32),
            pltpu.VMEM((d, 16), jnp.float32),
            pltpu.VMEM((n_tok, 1), jnp.int32),
            pltpu.VMEM((e_per, d, h), jnp.float32),
            pltpu.VMEM((d, h), jnp.float32),
            pltpu.VMEM((N_DEV, half, d, h), jnp.bfloat16),
            pltpu.VMEM((N_DEV, half, d, h), jnp.bfloat16),
            pltpu.SemaphoreType.DMA((N_DEV - 1, half)),
            pltpu.SemaphoreType.DMA((N_DEV - 1, half)),
            pltpu.SemaphoreType.DMA((N_DEV - 1, half)),
            pltpu.SemaphoreType.DMA((N_DEV - 1, half)),
            pltpu.SemaphoreType.DMA((5,)),
        ],
        compiler_params=pltpu.CompilerParams(collective_id=0),
    )(x, router_W, route_idx, expert_W, shared_W)


# device time: 29949 ns/iter; 1.0169x vs baseline; 1.0169x over previous
import jax
import jax.numpy as jnp
from jax import lax
from jax.experimental import pallas as pl
from jax.experimental.pallas import tpu as pltpu

N_DEV = 4


def kernel(x, router_W, route_idx, expert_W, shared_W):
    n_tok, d = x.shape
    e_per, _, h = expert_W.shape
    half = e_per // 2

    def body(x_ref, rw_ref, idx_ref, ew_ref, sw_ref, out_ref,
             cw_ref, ccw_ref, cw_send, cw_recv, ccw_send, ccw_recv):
        my = lax.axis_index("i")
        left = lax.rem(my - 1 + N_DEV, N_DEV)
        right = lax.rem(my + 1, N_DEV)

        barrier = pltpu.get_barrier_semaphore()
        for nbr in (left, right):
            pl.semaphore_signal(barrier, inc=1, device_id=(nbr,),
                                device_id_type=pl.DeviceIdType.MESH)
        pl.semaphore_wait(barrier, 2)

        rdma_cw = [[None] * half for _ in range(N_DEV - 1)]
        rdma_ccw = [[None] * half for _ in range(N_DEV - 1)]

        def start_cw(hop, sub):
            rdma_cw[hop][sub] = pltpu.make_async_remote_copy(
                src_ref=cw_ref.at[hop, sub],
                dst_ref=cw_ref.at[hop + 1, sub],
                send_sem=cw_send.at[hop, sub],
                recv_sem=cw_recv.at[hop, sub],
                device_id=(right,),
                device_id_type=pl.DeviceIdType.MESH,
            )
            rdma_cw[hop][sub].start()

        def start_ccw(hop, sub):
            rdma_ccw[hop][sub] = pltpu.make_async_remote_copy(
                src_ref=ccw_ref.at[hop, sub],
                dst_ref=ccw_ref.at[hop + 1, sub],
                send_sem=ccw_send.at[hop, sub],
                recv_sem=ccw_recv.at[hop, sub],
                device_id=(left,),
                device_id_type=pl.DeviceIdType.MESH,
            )
            rdma_ccw[hop][sub].start()

        cw_ref[0] = ew_ref[:half].astype(jnp.bfloat16)
        ccw_ref[0] = ew_ref[half:].astype(jnp.bfloat16)

        for sub in range(half):
            start_cw(0, sub)
            start_ccw(0, sub)

        xv = x_ref[...]
        idx = idx_ref[...]
        scores = jnp.dot(xv, rw_ref[...], preferred_element_type=jnp.float32)
        s_max = jnp.max(scores, axis=-1, keepdims=True)
        e_s = jnp.exp(scores - s_max)
        probs = e_s / jnp.sum(e_s, axis=-1, keepdims=True)
        iota = lax.broadcasted_iota(jnp.int32, scores.shape, 1)
        p_sel = jnp.sum(jnp.where(iota == idx, probs, 0.0), axis=-1,
                        keepdims=True)

        def expert_mm(acc, e_id, w):
            sel = jnp.where(idx == e_id, p_sel, 0.0)
            y = jnp.dot((xv * sel).astype(jnp.bfloat16), w,
                        preferred_element_type=jnp.float32)
            return y if acc is None else acc + y

        acc = None
        for hop in range(N_DEV):
            o_cw = lax.rem(my - hop + N_DEV, N_DEV)
            o_ccw = lax.rem(my + hop, N_DEV)
            for sub in range(half):
                if hop > 0:
                    rdma_cw[hop - 1][sub].wait_recv()
                    rdma_ccw[hop - 1][sub].wait_recv()
                    if hop < N_DEV - 1:
                        start_cw(hop, sub)
                        start_ccw(hop, sub)
                acc = expert_mm(acc, o_cw * e_per + sub, cw_ref[hop, sub])
                acc = expert_mm(acc, o_ccw * e_per + half + sub,
                                ccw_ref[hop, sub])
            if hop == 1:
                acc = acc + jnp.dot(xv, sw_ref[...],
                                    preferred_element_type=jnp.float32)
            if hop > 0:
                for sub in range(half):
                    rdma_cw[hop - 1][sub].wait_send()
                    rdma_ccw[hop - 1][sub].wait_send()

        out_ref[...] = acc

    return pl.pallas_call(
        body,
        out_shape=jax.ShapeDtypeStruct((n_tok, h), jnp.float32),
        in_specs=[pl.BlockSpec(memory_space=pltpu.VMEM)] * 5,
        out_specs=pl.BlockSpec(memory_space=pltpu.VMEM),
        scratch_shapes=[
            pltpu.VMEM((N_DEV, half, d, h), jnp.bfloat16),
            pltpu.VMEM((N_DEV, half, d, h), jnp.bfloat16),
            pltpu.SemaphoreType.DMA((N_DEV - 1, half)),
            pltpu.SemaphoreType.DMA((N_DEV - 1, half)),
            pltpu.SemaphoreType.DMA((N_DEV - 1, half)),
            pltpu.SemaphoreType.DMA((N_DEV - 1, half)),
        ],
        compiler_params=pltpu.CompilerParams(collective_id=0),
    )(x, router_W, route_idx, expert_W, shared_W)
